# baseline (speedup 1.0000x reference)
import jax
import jax.numpy as jnp
from jax import lax
from jax.experimental import pallas as pl

_B, _N, _D, _K = 16, 576, 64, 1024
_T = _B * _N
_BROWS = 8
_BLK = _BROWS * _N
_G = _B // _BROWS


def _next_f32(t):
    return lax.bitcast_convert_type(
        lax.bitcast_convert_type(t, jnp.uint32) + jnp.uint32(1), jnp.float32)


def _vq_body(x_ref, w_ref, idx_ref, q_ref):
    x = x_ref[...].reshape(_BLK, _D)
    w = w_ref[...]
    x2 = jnp.sum(x * x, axis=1, keepdims=True)
    w2 = jnp.sum(w * w, axis=1)
    ndot = lax.dot_general(-2.0 * x, w, (((1,), (1,)), ((), ())),
                           preferred_element_type=jnp.float32)
    d2 = (x2 + w2[None, :]) + ndot
    m = jnp.maximum(jnp.min(d2, axis=1, keepdims=True), 0.0)
    mt = lax.transpose(m, (1, 0))
    u = jnp.sqrt(mt)
    t = mt
    for _ in range(5):
        t1 = _next_f32(t)
        t = jnp.where(jnp.sqrt(t1) <= u, t1, t)
    tau = lax.transpose(t, (1, 0))
    ii = lax.broadcasted_iota(jnp.int32, d2.shape, 1).astype(jnp.float32)
    cand = jnp.where(d2 <= tau, ii, float(_K))
    idxf = jnp.min(cand, axis=1, keepdims=True)
    idx_ref[...] = idxf
    onehot = jnp.where(ii == idxf, 1.0, 0.0)
    q = lax.dot_general(onehot, w, (((1,), (0,)), ((), ())),
                        preferred_element_type=jnp.float32)
    q_ref[...] = q.reshape(_BROWS, _N, _D)


_vq_call = pl.pallas_call(
    _vq_body,
    grid=(_G,),
    in_specs=[
        pl.BlockSpec((_BROWS, _N, _D), lambda i: (i, 0, 0)),
        pl.BlockSpec((_K, _D), lambda i: (0, 0)),
    ],
    out_specs=[
        pl.BlockSpec((_BLK, 1), lambda i: (i, 0)),
        pl.BlockSpec((_BROWS, _N, _D), lambda i: (i, 0, 0)),
    ],
    out_shape=[
        jax.ShapeDtypeStruct((_T, 1), jnp.float32),
        jax.ShapeDtypeStruct((_B, _N, _D), jnp.float32),
    ],
)


def kernel(x, W):
    idxf, quantized = _vq_call(x, W)
    idx = idxf.reshape(_B, _N).astype(jnp.int32)
    return quantized, idx
